# baseline (speedup 1.0000x reference)
import jax
import jax.numpy as jnp
from jax import lax
from jax.experimental import pallas as pl
from jax.experimental.pallas import tpu as pltpu
from jax.experimental.pallas import tpu_sc as plsc

B = 16384
D = 192
H = 768
NC = 2
NS = 16
NW = NC * NS
BPW = B // NW
CH = 256


def _gather_a_body(cat_i, brand_i, item_i, seller_i,
                   t_cat, t_brand, t_item, t_seller,
                   o_c, o_b, o_i, o_s,
                   idx_v, buf, sem):
    wid = lax.axis_index("s") * NC + lax.axis_index("c")
    base = wid * BPW
    for idx_hbm, tab, out in ((cat_i, t_cat, o_c), (brand_i, t_brand, o_b),
                              (item_i, t_item, o_i), (seller_i, t_seller, o_s)):
        pltpu.sync_copy(idx_hbm.at[pl.ds(base, BPW)], idx_v)
        for ch in range(BPW // CH):
            lo = ch * CH
            ids = idx_v.at[pl.ds(lo, CH)]
            pltpu.async_copy(tab.at[ids, pl.ds(0, 128)], buf, sem).wait()
            pltpu.sync_copy(buf, out.at[pl.ds(base + lo, CH)])


def _gather_b_body(cat_i, brand_i, item_i, seller_i,
                   r_cat, r_brand, r_item, r_seller,
                   o_c, o_b, o_i, o_s,
                   idx_v, buf, sem):
    wid = lax.axis_index("s") * NC + lax.axis_index("c")
    base = wid * BPW
    for idx_hbm, tail, out in ((cat_i, r_cat, o_c), (brand_i, r_brand, o_b),
                               (item_i, r_item, o_i), (seller_i, r_seller, o_s)):
        pltpu.sync_copy(idx_hbm.at[pl.ds(base, BPW)], idx_v)
        for ch in range(BPW // CH):
            lo = ch * CH
            ids = idx_v.at[pl.ds(lo, CH)]
            pltpu.async_copy(tail.at[ids], buf, sem).wait()
            pltpu.sync_copy(buf, out.at[pl.ds(base + lo, CH)])


def _make_gather(body):
    return pl.kernel(
        body,
        mesh=plsc.VectorSubcoreMesh(core_axis_name="c", subcore_axis_name="s"),
        out_type=[jax.ShapeDtypeStruct((B, 128), jnp.float32)] * 4,
        scratch_types=[
            pltpu.VMEM((BPW,), jnp.int32),
            pltpu.VMEM((CH, 128), jnp.float32),
            pltpu.SemaphoreType.DMA,
        ],
    )


_gather_a = _make_gather(_gather_a_body)
_gather_b = _make_gather(_gather_b_body)


def _tails_body(x, out):
    t = x[:, 128:]
    out[...] = jnp.concatenate([t, t], axis=-1)


def _tails(emb):
    v = emb.shape[0]
    rt = 2000 if v % 2000 == 0 else v
    return pl.pallas_call(
        _tails_body,
        grid=(v // rt,),
        in_specs=[pl.BlockSpec((rt, D), lambda i: (i, 0))],
        out_specs=pl.BlockSpec((rt, 128), lambda i: (i, 0)),
        out_shape=jax.ShapeDtypeStruct((v, 128), jnp.float32),
    )(emb)


BM = 1024


def _mlp_body(ca, cb, ba, bb, ia, ib, sa, sb, w1, b1, w2, b2, out):
    x = jnp.concatenate(
        [ca[...], cb[...], ba[...], bb[...],
         ia[...], ib[...], sa[...], sb[...]], axis=-1)
    h = jnp.maximum(
        jnp.dot(x, w1[...], preferred_element_type=jnp.float32) + b1[...], 0.0)
    out[...] = jnp.dot(h, w2[...], preferred_element_type=jnp.float32) + b2[...]


_mlp = pl.pallas_call(
    _mlp_body,
    grid=(B // BM,),
    in_specs=[pl.BlockSpec((BM, 128), lambda i: (i, 0)) for _ in range(8)] + [
        pl.BlockSpec((8 * 128, H), lambda i: (0, 0)),
        pl.BlockSpec((1, H), lambda i: (0, 0)),
        pl.BlockSpec((H, H), lambda i: (0, 0)),
        pl.BlockSpec((1, H), lambda i: (0, 0)),
    ],
    out_specs=pl.BlockSpec((BM, H), lambda i: (i, 0)),
    out_shape=jax.ShapeDtypeStruct((B, H), jnp.float32),
)


def kernel(category, brand, item_id, seller,
           emb_category, emb_brand, emb_item_id, emb_seller,
           W1, b1, W2, b2):
    idx = [x.astype(jnp.int32) for x in (category, brand, item_id, seller)]
    pa = _gather_a(*idx, emb_category, emb_brand, emb_item_id, emb_seller)
    tails = [_tails(e) for e in
             (emb_category, emb_brand, emb_item_id, emb_seller)]
    pb = _gather_b(*idx, *tails)
    w1z = jnp.pad(W1.reshape(4, D, H), ((0, 0), (0, 64), (0, 0)))
    w1z = w1z.reshape(4 * 256, H)
    parts = [pa[0], pb[0], pa[1], pb[1], pa[2], pb[2], pa[3], pb[3]]
    return _mlp(*parts, w1z, b1.reshape(1, H), W2, b2.reshape(1, H))
